# baseline (speedup 1.0000x reference)
import functools

import jax
import jax.numpy as jnp
from jax import lax
from jax.experimental import pallas as pl
from jax.experimental.pallas import tpu as pltpu
from jax.experimental.pallas import tpu_sc as plsc

GTF = 3
L = 16
NC, NS = 2, 16
NW = NC * NS
CH = 128
K1_PLANES = 6
K2_PLANES = 3

_COMMON = dict(zip(["B", "S", "P"], [None] * 3))


def _worker_prolog(S, per_w, nch, pidx_hbm, seed_hbm, vote_hbm,
                   idx_v, seed_v, vote_v, xyz_sem):
    cid = lax.axis_index("c")
    sid = lax.axis_index("s")
    wid = sid * NC + cid
    base = wid * per_w
    b = wid // (S // per_w)
    s0 = (wid % (S // per_w)) * per_w
    for c in range(nch):
        pltpu.sync_copy(pidx_hbm.at[b, pl.ds(s0 + c * CH, CH)], idx_v.at[c])
    xyz_descs = []
    for k in range(3):
        xyz_descs.append(pltpu.async_copy(
            seed_hbm.at[k, pl.ds(base, per_w)], seed_v.at[k], xyz_sem))
        xyz_descs.append(pltpu.async_copy(
            vote_hbm.at[k, pl.ds(base, per_w)], vote_v.at[k], xyz_sem))
    return wid, base, b, xyz_descs


def _dist_block(gt_v, seed_v, vote_v, sl, planes):
    sx = [seed_v[k, sl] for k in range(3)]
    vx = [vote_v[k, sl] for k in range(3)]
    d = None
    for j in range(len(planes) // 3):
        dj = None
        for k in range(3):
            t = jnp.abs(vx[k] - (gt_v[3 * j + k, sl] + sx[k]))
            dj = t if dj is None else dj + t
        d = dj if d is None else jnp.minimum(d, dj)
    return d


def _make_k1(B, S, P):
    N = B * S
    per_w = N // NW
    nch = per_w // CH
    mesh = plsc.VectorSubcoreMesh(core_axis_name="c", subcore_axis_name="s")

    @functools.partial(
        pl.kernel,
        mesh=mesh,
        compiler_params=pltpu.CompilerParams(
            needs_layout_passes=False, use_tc_tiling_on_sc=False),
        out_type=[
            jax.ShapeDtypeStruct((NW, per_w), jnp.float32),
            jax.ShapeDtypeStruct((NW, per_w), jnp.float32),
        ],
        scratch_types=[
            pltpu.VMEM((nch, CH), jnp.int32),
            pltpu.VMEM((K1_PLANES, per_w), jnp.float32),
            pltpu.VMEM((per_w,), jnp.int32),
            pltpu.VMEM((3, per_w), jnp.float32),
            pltpu.VMEM((3, per_w), jnp.float32),
            pltpu.VMEM((per_w,), jnp.float32),
            pltpu.VMEM((per_w,), jnp.float32),
            pltpu.SemaphoreType.DMA,
            pltpu.SemaphoreType.DMA,
        ],
    )
    def k1(pidx_hbm, seed_hbm, vote_hbm, vla_hbm, mask_hbm,
           d01_hbm, mf_hbm,
           idx_v, gt_v, mask_v, seed_v, vote_v, d_buf, m_buf, sem, xyz_sem):
        wid, base, b, xyz_descs = _worker_prolog(
            S, per_w, nch, pidx_hbm, seed_hbm, vote_hbm,
            idx_v, seed_v, vote_v, xyz_sem)

        descs = []
        for c in range(nch):
            dst = pl.ds(c * CH, CH)
            idx_c = idx_v.at[c]
            chunk = [pltpu.async_copy(
                mask_hbm.at[b].at[idx_c], mask_v.at[dst], sem)]
            for k in range(K1_PLANES):
                chunk.append(pltpu.async_copy(
                    vla_hbm.at[k, b].at[idx_c], gt_v.at[k, dst], sem))
            descs.append(chunk)

        for dsc in xyz_descs:
            dsc.wait()

        def body(c, g, _):
            sl = pl.ds(c * CH + g * L, L)
            d = _dist_block(gt_v, seed_v, vote_v, sl, range(K1_PLANES))
            d_buf[sl] = d
            m_buf[sl] = mask_v[sl].astype(jnp.float32)
            return 0

        for c in range(nch):
            for dsc in descs[c]:
                dsc.wait()
            lax.fori_loop(0, CH // L, functools.partial(body, c), 0)

        pltpu.sync_copy(d_buf, d01_hbm.at[wid])
        pltpu.sync_copy(m_buf, mf_hbm.at[wid])

    return k1


def _make_k2(B, S, P):
    N = B * S
    per_w = N // NW
    nch = per_w // CH
    mesh = plsc.VectorSubcoreMesh(core_axis_name="c", subcore_axis_name="s")

    @functools.partial(
        pl.kernel,
        mesh=mesh,
        compiler_params=pltpu.CompilerParams(
            needs_layout_passes=False, use_tc_tiling_on_sc=False),
        out_type=[
            jax.ShapeDtypeStruct((NW, L), jnp.float32),
            jax.ShapeDtypeStruct((NW, L), jnp.float32),
        ],
        scratch_types=[
            pltpu.VMEM((nch, CH), jnp.int32),
            pltpu.VMEM((K2_PLANES, per_w), jnp.float32),
            pltpu.VMEM((per_w,), jnp.float32),
            pltpu.VMEM((per_w,), jnp.float32),
            pltpu.VMEM((3, per_w), jnp.float32),
            pltpu.VMEM((3, per_w), jnp.float32),
            pltpu.VMEM((L,), jnp.float32),
            pltpu.VMEM((L,), jnp.float32),
            pltpu.SemaphoreType.DMA,
            pltpu.SemaphoreType.DMA,
        ],
    )
    def k2(pidx_hbm, seed_hbm, vote_hbm, vlb_hbm, d01_hbm, mf_hbm,
           num_hbm, den_hbm,
           idx_v, gt_v, d01_v, mf_v, seed_v, vote_v,
           accn_v, accd_v, sem, xyz_sem):
        wid, base, b, xyz_descs = _worker_prolog(
            S, per_w, nch, pidx_hbm, seed_hbm, vote_hbm,
            idx_v, seed_v, vote_v, xyz_sem)
        xyz_descs.append(pltpu.async_copy(d01_hbm.at[wid], d01_v, xyz_sem))
        xyz_descs.append(pltpu.async_copy(mf_hbm.at[wid], mf_v, xyz_sem))

        descs = []
        for c in range(nch):
            dst = pl.ds(c * CH, CH)
            idx_c = idx_v.at[c]
            descs.append([pltpu.async_copy(
                vlb_hbm.at[k, b].at[idx_c], gt_v.at[k, dst], sem)
                for k in range(K2_PLANES)])

        for dsc in xyz_descs:
            dsc.wait()

        zeros = jnp.zeros((L,), jnp.float32)

        def body(c, g, carry):
            num, den = carry
            sl = pl.ds(c * CH + g * L, L)
            d2 = _dist_block(gt_v, seed_v, vote_v, sl, range(K2_PLANES))
            d = jnp.minimum(d01_v[sl], d2)
            mf = mf_v[sl]
            return num + d * mf, den + mf

        acc = (zeros, zeros)
        for c in range(nch):
            for dsc in descs[c]:
                dsc.wait()
            acc = lax.fori_loop(0, CH // L, functools.partial(body, c), acc)
        num, den = acc

        accn_v[...] = num
        accd_v[...] = den
        pltpu.sync_copy(accn_v, num_hbm.at[wid])
        pltpu.sync_copy(accd_v, den_hbm.at[wid])

    return k2


def kernel(seed_xyz, vote_xyz, seed_inds, vote_label_mask, vote_label):
    B, S, _ = seed_xyz.shape
    P = vote_label.shape[1]
    N = B * S

    seed_t = jnp.transpose(seed_xyz, (2, 0, 1)).reshape(3, N)
    vote_t = jnp.transpose(vote_xyz, (2, 0, 1)).reshape(3, N)
    vl_t = jnp.transpose(vote_label, (2, 0, 1))
    vla = vl_t[:K1_PLANES]
    vlb = vl_t[K1_PLANES:]
    pidx = seed_inds.astype(jnp.int32)
    mask2 = vote_label_mask.astype(jnp.int32)

    d01, mf = _make_k1(B, S, P)(pidx, seed_t, vote_t, vla, mask2)
    num, den = _make_k2(B, S, P)(pidx, seed_t, vote_t, vlb, d01, mf)
    return jnp.sum(num) / (jnp.sum(den) + 1e-6)


# speedup vs baseline: 1.3101x; 1.3101x over previous
import functools

import jax
import jax.numpy as jnp
from jax import lax
from jax.experimental import pallas as pl
from jax.experimental.pallas import tpu as pltpu
from jax.experimental.pallas import tpu_sc as plsc

GTF = 3
NCOMP = GTF * 3
L = 16
NC, NS = 2, 16
NW = NC * NS
CH = 128


def _make_sc_kernel(B, S, P):
    N = B * S
    assert N % NW == 0
    per_w = N // NW
    assert per_w % CH == 0
    nch = per_w // CH
    BP = B * P

    mesh = plsc.VectorSubcoreMesh(core_axis_name="c", subcore_axis_name="s")

    @functools.partial(
        pl.kernel,
        mesh=mesh,
        compiler_params=pltpu.CompilerParams(
            needs_layout_passes=False, use_tc_tiling_on_sc=False),
        out_type=[
            jax.ShapeDtypeStruct((NW, L), jnp.float32),
            jax.ShapeDtypeStruct((NW, L), jnp.float32),
        ],
        scratch_types=[
            pltpu.VMEM((nch, CH), jnp.int32),
            pltpu.VMEM((NCOMP * nch, CH), jnp.int32),
            pltpu.VMEM((NCOMP, per_w), jnp.float32),
            pltpu.VMEM((per_w,), jnp.int32),
            pltpu.VMEM((3, per_w), jnp.float32),
            pltpu.VMEM((3, per_w), jnp.float32),
            pltpu.VMEM((L,), jnp.float32),
            pltpu.VMEM((L,), jnp.float32),
            pltpu.SemaphoreType.DMA,
            pltpu.SemaphoreType.DMA,
        ],
    )
    def sc_kernel(idx_hbm, seed_hbm, vote_hbm, vl_hbm, mask_hbm,
                  num_hbm, den_hbm,
                  idx_v, idx9_v, gt_v, mask_v, seed_v, vote_v,
                  accn_v, accd_v, sem, xyz_sem):
        cid = lax.axis_index("c")
        sid = lax.axis_index("s")
        wid = sid * NC + cid
        base = wid * per_w

        pltpu.sync_copy(idx_hbm.at[pl.ds(wid * nch, nch)], idx_v)
        xyz_descs = []
        for k in range(3):
            xyz_descs.append(pltpu.async_copy(
                seed_hbm.at[k, pl.ds(base, per_w)], seed_v.at[k], xyz_sem))
            xyz_descs.append(pltpu.async_copy(
                vote_hbm.at[k, pl.ds(base, per_w)], vote_v.at[k], xyz_sem))

        descs = []
        for c in range(nch):
            for s in range(CH // L):
                idx16 = idx_v[c, pl.ds(s * L, L)]
                for k in range(NCOMP):
                    idx9_v[k * nch + c, pl.ds(s * L, L)] = idx16 + k * BP
            dst = pl.ds(c * CH, CH)
            chunk_descs = [pltpu.async_copy(
                mask_hbm.at[idx_v.at[c]], mask_v.at[dst], sem)]
            for k in range(NCOMP):
                chunk_descs.append(pltpu.async_copy(
                    vl_hbm.at[idx9_v.at[k * nch + c]], gt_v.at[k, dst], sem))
            descs.append(chunk_descs)

        for dsc in xyz_descs:
            dsc.wait()

        zeros = jnp.zeros((L,), jnp.float32)

        def body(c, g, carry):
            num, den = carry
            sl = pl.ds(c * CH + g * L, L)
            sx = [seed_v[k, sl] for k in range(3)]
            vx = [vote_v[k, sl] for k in range(3)]
            d = None
            for j in range(GTF):
                dj = None
                for k in range(3):
                    t = jnp.abs(vx[k] - (gt_v[3 * j + k, sl] + sx[k]))
                    dj = t if dj is None else dj + t
                d = dj if d is None else jnp.minimum(d, dj)
            mf = mask_v[sl].astype(jnp.float32)
            return num + d * mf, den + mf

        acc = (zeros, zeros)
        for c in range(nch):
            for dsc in descs[c]:
                dsc.wait()
            acc = lax.fori_loop(
                0, CH // L, functools.partial(body, c), acc)
        num, den = acc

        accn_v[...] = num
        accd_v[...] = den
        pltpu.sync_copy(accn_v, num_hbm.at[wid])
        pltpu.sync_copy(accd_v, den_hbm.at[wid])

    return sc_kernel


def kernel(seed_xyz, vote_xyz, seed_inds, vote_label_mask, vote_label):
    B, S, _ = seed_xyz.shape
    P = vote_label.shape[1]
    N = B * S

    idx_g = (seed_inds.astype(jnp.int32)
             + (jnp.arange(B, dtype=jnp.int32) * P)[:, None])
    idx_g = idx_g.reshape(N // CH, CH)
    seed_t = jnp.transpose(seed_xyz, (2, 0, 1)).reshape(3, N)
    vote_t = jnp.transpose(vote_xyz, (2, 0, 1)).reshape(3, N)
    vl_t = jnp.transpose(vote_label, (2, 0, 1)).reshape(NCOMP * B * P)
    mask_flat = vote_label_mask.astype(jnp.int32).reshape(B * P)

    sc = _make_sc_kernel(B, S, P)
    num, den = sc(idx_g, seed_t, vote_t, vl_t, mask_flat)
    return jnp.sum(num) / (jnp.sum(den) + 1e-6)
